# baseline (speedup 1.0000x reference)
import functools

import jax
import jax.numpy as jnp
from jax import lax
from jax.experimental import pallas as pl
from jax.experimental.pallas import tpu as pltpu

_SMOOTH = 0.1
_NSTREAM = 4
_BLOCK_ROWS = 1024


def _stream_part(x, t, classes):
    m = jnp.max(x, axis=1, keepdims=True)
    se = jnp.sum(jnp.exp(x - m), axis=1)
    sum_lse = jnp.sum(m) + jnp.sum(jnp.log(se))

    col = lax.broadcasted_iota(jnp.int32, x.shape, 1)
    a = _SMOOTH / (classes - 1)
    b = (1.0 - _SMOOTH) - a
    wx = a * jnp.sum(x) + b * jnp.sum(jnp.where(col == t[:, None], x, 0.0))
    return (a * classes + b) * sum_lse - wx


def _tc_body(*refs, classes):
    out_ref = refs[-1]
    x_refs = refs[:_NSTREAM]
    t_refs = refs[_NSTREAM:-1]
    i = pl.program_id(0)

    part = _stream_part(x_refs[0][...], t_refs[0][0, 0, :], classes)
    for xr, tr in zip(x_refs[1:], t_refs[1:]):
        part += _stream_part(xr[...], tr[0, 0, :], classes)

    @pl.when(i == 0)
    def _init():
        out_ref[0, 0] = 0.0

    out_ref[0, 0] += part


def kernel(prediction, target):
    n, classes = prediction.shape
    grid = n // _NSTREAM // _BLOCK_ROWS
    tgt = target.astype(jnp.int32).reshape(n // _BLOCK_ROWS, 1, _BLOCK_ROWS)

    x_specs = [
        pl.BlockSpec((_BLOCK_ROWS, classes),
                     lambda i, s=s, g=grid: (i + s * g, 0))
        for s in range(_NSTREAM)
    ]
    t_specs = [
        pl.BlockSpec((1, 1, _BLOCK_ROWS),
                     lambda i, s=s, g=grid: (i + s * g, 0, 0))
        for s in range(_NSTREAM)
    ]
    total = pl.pallas_call(
        functools.partial(_tc_body, classes=classes),
        grid=(grid,),
        in_specs=x_specs + t_specs,
        out_specs=pl.BlockSpec(
            (1, 1), lambda i: (0, 0), memory_space=pltpu.SMEM
        ),
        out_shape=jax.ShapeDtypeStruct((1, 1), jnp.float32),
    )(*([prediction] * _NSTREAM + [tgt] * _NSTREAM))

    return total[0, 0] / n


# speedup vs baseline: 1.0045x; 1.0045x over previous
import functools

import jax
import jax.numpy as jnp
from jax import lax
from jax.experimental import pallas as pl
from jax.experimental.pallas import tpu as pltpu

_SMOOTH = 0.1
_NSTREAM = 4
_BLOCK_ROWS = 512


def _stream_part(x, t, classes):
    m = jnp.max(x, axis=1, keepdims=True)
    se = jnp.sum(jnp.exp(x - m), axis=1)
    sum_lse = jnp.sum(m) + jnp.sum(jnp.log(se))

    col = lax.broadcasted_iota(jnp.int32, x.shape, 1)
    a = _SMOOTH / (classes - 1)
    b = (1.0 - _SMOOTH) - a
    wx = a * jnp.sum(x) + b * jnp.sum(jnp.where(col == t[:, None], x, 0.0))
    return (a * classes + b) * sum_lse - wx


def _tc_body(*refs, classes):
    out_ref = refs[-1]
    x_refs = refs[:_NSTREAM]
    t_refs = refs[_NSTREAM:-1]
    i = pl.program_id(0)

    part = _stream_part(x_refs[0][...], t_refs[0][0, 0, :], classes)
    for xr, tr in zip(x_refs[1:], t_refs[1:]):
        part += _stream_part(xr[...], tr[0, 0, :], classes)

    @pl.when(i == 0)
    def _init():
        out_ref[0, 0] = 0.0

    out_ref[0, 0] += part


def kernel(prediction, target):
    n, classes = prediction.shape
    grid = n // _NSTREAM // _BLOCK_ROWS
    tgt = target.astype(jnp.int32).reshape(n // _BLOCK_ROWS, 1, _BLOCK_ROWS)

    x_specs = [
        pl.BlockSpec((_BLOCK_ROWS, classes),
                     lambda i, s=s, g=grid: (i + s * g, 0))
        for s in range(_NSTREAM)
    ]
    t_specs = [
        pl.BlockSpec((1, 1, _BLOCK_ROWS),
                     lambda i, s=s, g=grid: (i + s * g, 0, 0))
        for s in range(_NSTREAM)
    ]
    total = pl.pallas_call(
        functools.partial(_tc_body, classes=classes),
        grid=(grid,),
        in_specs=x_specs + t_specs,
        out_specs=pl.BlockSpec(
            (1, 1), lambda i: (0, 0), memory_space=pltpu.SMEM
        ),
        out_shape=jax.ShapeDtypeStruct((1, 1), jnp.float32),
    )(*([prediction] * _NSTREAM + [tgt] * _NSTREAM))

    return total[0, 0] / n


# speedup vs baseline: 1.0222x; 1.0176x over previous
import functools

import jax
import jax.numpy as jnp
from jax import lax
from jax.experimental import pallas as pl
from jax.experimental.pallas import tpu as pltpu

_SMOOTH = 0.1
_NSTREAM = 2
_BLOCK_ROWS = 1024


def _stream_part(x, t, classes):
    m = jnp.max(x, axis=1, keepdims=True)
    se = jnp.sum(jnp.exp(x - m), axis=1)
    sum_lse = jnp.sum(m) + jnp.sum(jnp.log(se))

    col = lax.broadcasted_iota(jnp.int32, x.shape, 1)
    a = _SMOOTH / (classes - 1)
    b = (1.0 - _SMOOTH) - a
    wx = a * jnp.sum(x) + b * jnp.sum(jnp.where(col == t[:, None], x, 0.0))
    return (a * classes + b) * sum_lse - wx


def _tc_body(*refs, classes):
    out_ref = refs[-1]
    x_refs = refs[:_NSTREAM]
    t_refs = refs[_NSTREAM:-1]
    i = pl.program_id(0)

    part = _stream_part(x_refs[0][...], t_refs[0][0, 0, :], classes)
    for xr, tr in zip(x_refs[1:], t_refs[1:]):
        part += _stream_part(xr[...], tr[0, 0, :], classes)

    @pl.when(i == 0)
    def _init():
        out_ref[0, 0] = 0.0

    out_ref[0, 0] += part


def kernel(prediction, target):
    n, classes = prediction.shape
    grid = n // _NSTREAM // _BLOCK_ROWS
    tgt = target.astype(jnp.int32).reshape(n // _BLOCK_ROWS, 1, _BLOCK_ROWS)

    x_specs = [
        pl.BlockSpec((_BLOCK_ROWS, classes),
                     lambda i, s=s, g=grid: (i + s * g, 0))
        for s in range(_NSTREAM)
    ]
    t_specs = [
        pl.BlockSpec((1, 1, _BLOCK_ROWS),
                     lambda i, s=s, g=grid: (i + s * g, 0, 0))
        for s in range(_NSTREAM)
    ]
    total = pl.pallas_call(
        functools.partial(_tc_body, classes=classes),
        grid=(grid,),
        in_specs=x_specs + t_specs,
        out_specs=pl.BlockSpec(
            (1, 1), lambda i: (0, 0), memory_space=pltpu.SMEM
        ),
        out_shape=jax.ShapeDtypeStruct((1, 1), jnp.float32),
    )(*([prediction] * _NSTREAM + [tgt] * _NSTREAM))

    return total[0, 0] / n
